# baseline (speedup 1.0000x reference)
import functools

import jax
import jax.numpy as jnp
from jax import lax
from jax.experimental import pallas as pl
from jax.experimental.pallas import tpu as pltpu
from jax.experimental.pallas import tpu_sc as plsc

D = 128
DE = 16
H = 64
NEG = -1e30
WORKERS = 32
LANES = 16


@functools.lru_cache(maxsize=None)
def _build(n, e):
    nblk = n // 5 if n % 5 == 0 else n // 8
    while n % nblk or nblk % 8:
        nblk //= 2
    eblk = 3200 if (e // 2) % 3200 == 0 else e // 2

    def a_body(x_ref, wi_ref, wo_ref, bi_ref, bo_ref, o_ref):
        xb = x_ref[...]
        dn = (((1,), (1,)), ((), ()))
        o_ref[0] = (lax.dot_general(xb, wi_ref[...], dn,
                                    preferred_element_type=jnp.float32)
                    + bi_ref[...]).astype(jnp.bfloat16)
        o_ref[1] = (lax.dot_general(xb, wo_ref[...], dn,
                                    preferred_element_type=jnp.float32)
                    + bo_ref[...]).astype(jnp.bfloat16)

    phase_a = pl.pallas_call(
        a_body,
        grid=(n // nblk,),
        in_specs=[pl.BlockSpec((nblk, D), lambda i: (i, 0)),
                  pl.BlockSpec((H, D), lambda i: (0, 0)),
                  pl.BlockSpec((H, D), lambda i: (0, 0)),
                  pl.BlockSpec((1, H), lambda i: (0, 0)),
                  pl.BlockSpec((1, H), lambda i: (0, 0))],
        out_specs=pl.BlockSpec((2, nblk, H), lambda i: (0, i, 0)),
        out_shape=jax.ShapeDtypeStruct((2, n, H), jnp.bfloat16),
    )

    eh = e // 2
    nb = eh // eblk
    nbp = ((nb + 7) // 8) * 8

    def _bsel_half(eat, r, c):
        lt = r < c
        ltf = lt.astype(jnp.float32).reshape(1, eblk)
        gtf = (r > c).astype(jnp.float32).reshape(1, eblk)
        am = jnp.concatenate([eat * ltf, eat * gtf], axis=0)
        return am, jnp.where(r == c, 2 * n, r + lt.astype(jnp.int32) * n)

    def b_body(eatl_ref, eatr_ref, rowl_ref, coll_ref, rowr_ref, colr_ref,
               wbig_ref, bsel_ref, gidxl_ref, gidxr_ref):
        i = pl.program_id(0)
        dn = (((0,), (1,)), ((), ()))
        wb = wbig_ref[...]
        aml, gl = _bsel_half(eatl_ref[...], rowl_ref[i % 8, :],
                             coll_ref[i % 8, :])
        amr, gr = _bsel_half(eatr_ref[...], rowr_ref[i % 8, :],
                             colr_ref[i % 8, :])
        bl = lax.dot_general(aml, wb, dn, preferred_element_type=jnp.float32)
        br = lax.dot_general(amr, wb, dn, preferred_element_type=jnp.float32)
        bsel_ref[...] = jnp.concatenate([bl, br], axis=1).astype(jnp.bfloat16)
        gidxl_ref[i % 8, :] = gl
        gidxr_ref[i % 8, :] = gr

    phase_b = pl.pallas_call(
        b_body,
        grid=(nb,),
        in_specs=[pl.BlockSpec((DE, eblk), lambda i: (0, i)),
                  pl.BlockSpec((DE, eblk), lambda i: (0, i + nb)),
                  pl.BlockSpec((8, eblk), lambda i: (i // 8, 0)),
                  pl.BlockSpec((8, eblk), lambda i: (i // 8, 0)),
                  pl.BlockSpec((8, eblk), lambda i: (i // 8, 0)),
                  pl.BlockSpec((8, eblk), lambda i: (i // 8, 0)),
                  pl.BlockSpec((H, 2 * DE), lambda i: (0, 0))],
        out_specs=[pl.BlockSpec((eblk, 2 * H), lambda i: (i, 0)),
                   pl.BlockSpec((8, eblk), lambda i: (i // 8, 0)),
                   pl.BlockSpec((8, eblk), lambda i: (i // 8, 0))],
        out_shape=[jax.ShapeDtypeStruct((eh, 2 * H), jnp.bfloat16),
                   jax.ShapeDtypeStruct((nbp, eblk), jnp.int32),
                   jax.ShapeDtypeStruct((nbp, eblk), jnp.int32)],
    )

    def c_body(p1_ref, p2_ref, w1_ref, w2_ref, bn_ref, o_ref):
        fi = p1_ref[0] + p1_ref[1]
        fo = p2_ref[0] + p2_ref[1]
        dn = (((1,), (1,)), ((), ()))
        o = lax.dot_general(fi, w1_ref[...], dn,
                            preferred_element_type=jnp.float32)
        o = o + lax.dot_general(fo, w2_ref[...], dn,
                                preferred_element_type=jnp.float32)
        o_ref[...] = jnp.maximum(o + bn_ref[...], 0.0)

    nsteps = n // nblk
    phase_c = pl.pallas_call(
        c_body,
        grid=(nsteps,),
        in_specs=[pl.BlockSpec((2, nblk, H), lambda i: (0, i, 0)),
                  pl.BlockSpec((2, nblk, H), lambda i: (0, i + nsteps, 0)),
                  pl.BlockSpec((D, H), lambda i: (0, 0)),
                  pl.BlockSpec((D, H), lambda i: (0, 0)),
                  pl.BlockSpec((1, D), lambda i: (0, 0))],
        out_specs=pl.BlockSpec((nblk, D), lambda i: (i, 0)),
        out_shape=jax.ShapeDtypeStruct((n, D), jnp.float32),
    )

    return phase_a, phase_b, phase_c


@functools.lru_cache(maxsize=None)
def _build_sc(n, e):
    ew = e // WORKERS
    bk = 80
    while ew % bk:
        bk -= 8
    nbk = ew // bk
    cs = 160
    nchunk = (2 * n) // cs

    mesh = plsc.VectorSubcoreMesh(core_axis_name="c", subcore_axis_name="s",
                                  num_cores=2, num_subcores=16)

    @functools.partial(
        pl.kernel, mesh=mesh,
        compiler_params=pltpu.CompilerParams(use_tc_tiling_on_sc=False,
                                             needs_layout_passes=False),
        out_type=jax.ShapeDtypeStruct((2, 2 * n, H), jnp.float32),
        scratch_types=[
            pltpu.VMEM((nbk, bk), jnp.int32),
            pltpu.VMEM((bk, H), jnp.bfloat16),
            pltpu.VMEM((bk, H), jnp.bfloat16),
            pltpu.VMEM((bk // 2, 2 * H), jnp.bfloat16),
            pltpu.VMEM((bk // 2, 2 * H), jnp.bfloat16),
            pltpu.VMEM((bk, H), jnp.float32),
            pltpu.VMEM((bk, H), jnp.float32),
            pltpu.VMEM((cs, H), jnp.float32),
            pltpu.VMEM_SHARED((2 * n + 8, H), jnp.float32),
            pltpu.SemaphoreType.DMA,
            pltpu.SemaphoreType.DMA,
            pltpu.SemaphoreType.DMA,
            pltpu.SemaphoreType.DMA,
            pltpu.SemaphoreType.DMA,
            pltpu.SemaphoreType.DMA,
        ],
    )
    def phase_sc(t_hbm, b_hbm, idx_hbm, z_hbm, out_hbm,
                 idx_all, g0, g1, b0, b1, r0, r1, bounce_v, acc_sh,
                 sg0, sg1, sb0, sb1, ss0, ss1):
        gath = (g0, g1)
        bsel = (b0, b1)
        res = (r0, r1)
        semg = (sg0, sg1)
        semb = (sb0, sb1)
        sems = (ss0, ss1)
        cid = lax.axis_index("c")
        sid = lax.axis_index("s")
        wid = sid * 2 + cid

        pltpu.sync_copy(idx_hbm.at[pl.ds(wid * nbk, nbk)], idx_all)

        pltpu.sync_copy(z_hbm, bounce_v)

        def zchunk(k, carry):
            @pl.when(sid == k % 16)
            def _():
                pltpu.sync_copy(bounce_v, acc_sh.at[pl.ds(k * cs, cs)])
            return carry

        lax.fori_loop(0, nchunk, zchunk, 0)
        plsc.subcore_barrier()

        def issue(s, p):
            base = (wid * ew + s * bk) // 2
            pltpu.async_copy(b_hbm.at[pl.ds(base, bk // 2)], bsel[p],
                             semb[p])
            pltpu.async_copy(t_hbm.at[idx_all.at[s]], gath[p], semg[p])

        def process(s, p):
            pltpu.make_async_copy(b_hbm.at[pl.ds(0, bk // 2)], bsel[p],
                                  semb[p]).wait()
            pltpu.make_async_copy(t_hbm.at[pl.ds(0, bk)], gath[p],
                                  semg[p]).wait()

            @pl.when(s >= 2)
            def _():
                pltpu.make_async_copy(t_hbm.at[pl.ds(0, bk)], res[p],
                                      sems[p]).wait()

            @plsc.parallel_loop(0, bk // 2, unroll=4)
            def _(ei):
                for j in range(H // 32):
                    sl = pl.ds(j * 32, 32)
                    slh = pl.ds(H + j * 32, 32)
                    v = jnp.maximum(gath[p][ei, sl] + bsel[p][ei, sl],
                                    jnp.bfloat16(0))
                    lo, hi = plsc.unpack(v, format=plsc.PackFormat.INTERLEAVED)
                    res[p][ei, pl.ds(j * 32, LANES)] = lo
                    res[p][ei, pl.ds(j * 32 + LANES, LANES)] = hi
                    v2 = jnp.maximum(
                        gath[p][bk // 2 + ei, sl] + bsel[p][ei, slh],
                        jnp.bfloat16(0))
                    lo2, hi2 = plsc.unpack(v2,
                                           format=plsc.PackFormat.INTERLEAVED)
                    res[p][bk // 2 + ei, pl.ds(j * 32, LANES)] = lo2
                    res[p][bk // 2 + ei, pl.ds(j * 32 + LANES, LANES)] = hi2
            pltpu.async_copy(res[p], acc_sh.at[idx_all.at[s]], sems[p],
                             add=True)

            @pl.when(s + 2 < nbk)
            def _():
                issue(s + 2, p)

        issue(0, 0)
        if nbk > 1:
            issue(1, 1)

        def pair(j, carry):
            for p in range(2):
                s = 2 * j + p

                @pl.when(s < nbk)
                def _():
                    process(s, p)
            return carry

        lax.fori_loop(0, (nbk + 1) // 2, pair, 0)
        pltpu.make_async_copy(t_hbm.at[pl.ds(0, bk)], res[0], sems[0]).wait()
        if nbk > 1:
            pltpu.make_async_copy(t_hbm.at[pl.ds(0, bk)], res[1],
                                  sems[1]).wait()
        plsc.subcore_barrier()

        def wchunk(k, carry):
            @pl.when(sid == k % 16)
            def _():
                pltpu.sync_copy(acc_sh.at[pl.ds(k * cs, cs)], bounce_v)
                pltpu.sync_copy(bounce_v, out_hbm.at[cid, pl.ds(k * cs, cs)])
            return carry

        lax.fori_loop(0, nchunk, wchunk, 0)

    return phase_sc, cs, bk


def kernel(x, edge_index, edge_attr, W_out, b_out, W_in, b_in, W_node,
           b_node):
    n = x.shape[0]
    e = edge_attr.shape[0]
    phase_a, phase_b, phase_c = _build(n, e)
    phase_sc, cs, bk = _build_sc(n, e)

    row = edge_index[0]
    col = edge_index[1]

    t = phase_a(x, W_in[:, :D], W_out[:, :D], b_in.reshape(1, H),
                b_out.reshape(1, H)).reshape(2 * n, H)
    t = jnp.concatenate([t, jnp.full((8, H), NEG, jnp.bfloat16)], axis=0)
    wbig = jnp.concatenate([W_out[:, D:], W_in[:, D:]], axis=1)
    eh = e // 2
    eblk = 3200 if eh % 3200 == 0 else eh
    nb = eh // eblk
    nbp = ((nb + 7) // 8) * 8

    def chunk2(v):
        return jnp.pad(v.reshape(nb, eblk), ((0, nbp - nb), (0, 0)))

    eat = edge_attr.T
    bsel2, gl2, gr2 = phase_b(eat, eat, chunk2(row[:eh]), chunk2(col[:eh]),
                              chunk2(row[eh:]), chunk2(col[eh:]), wbig)
    hbk = bk // 2
    gidx_sc = jnp.concatenate([gl2[:nb].reshape(eh // hbk, hbk),
                               gr2[:nb].reshape(eh // hbk, hbk)], axis=1)
    zeros = jnp.zeros((cs, H), jnp.float32)
    partials = phase_sc(t, bsel2, gidx_sc, zeros)
    perm = [g * 32 + (2 * r if r < 16 else 2 * (r - 16) + 1)
            for g in range(H // 32) for r in range(32)]
    return phase_c(partials, partials, W_node[:, :H][:, perm],
                   W_node[:, H:][:, perm],
                   b_node.reshape(1, D))


# speedup vs baseline: 1.2880x; 1.2880x over previous
import functools

import jax
import jax.numpy as jnp
from jax import lax
from jax.experimental import pallas as pl
from jax.experimental.pallas import tpu as pltpu
from jax.experimental.pallas import tpu_sc as plsc

D = 128
DE = 16
H = 64
NEG = -1e30
WORKERS = 32
LANES = 16


@functools.lru_cache(maxsize=None)
def _build(n, e):
    nblk = n // 5 if n % 5 == 0 else n // 8
    while n % nblk or nblk % 8:
        nblk //= 2
    eblk = 3200 if (e // 2) % 3200 == 0 else e // 2

    def a_body(x_ref, wi_ref, wo_ref, bi_ref, bo_ref, o_ref):
        xb = x_ref[...]
        dn = (((1,), (1,)), ((), ()))
        o_ref[0] = lax.dot_general(xb, wi_ref[...], dn,
                                   preferred_element_type=jnp.float32) \
            + bi_ref[...]
        o_ref[1] = lax.dot_general(xb, wo_ref[...], dn,
                                   preferred_element_type=jnp.float32) \
            + bo_ref[...]

    phase_a = pl.pallas_call(
        a_body,
        grid=(n // nblk,),
        in_specs=[pl.BlockSpec((nblk, D), lambda i: (i, 0)),
                  pl.BlockSpec((H, D), lambda i: (0, 0)),
                  pl.BlockSpec((H, D), lambda i: (0, 0)),
                  pl.BlockSpec((1, H), lambda i: (0, 0)),
                  pl.BlockSpec((1, H), lambda i: (0, 0))],
        out_specs=pl.BlockSpec((2, nblk, H), lambda i: (0, i, 0)),
        out_shape=jax.ShapeDtypeStruct((2, n, H), jnp.float32),
    )

    eh = e // 2
    nb = eh // eblk
    nbp = ((nb + 7) // 8) * 8

    def _bsel_half(eat, r, c):
        lt = r < c
        ltf = lt.astype(jnp.float32).reshape(1, eblk)
        gtf = (r > c).astype(jnp.float32).reshape(1, eblk)
        am = jnp.concatenate([eat * ltf, eat * gtf], axis=0)
        return am, jnp.where(r == c, 2 * n, r + lt.astype(jnp.int32) * n)

    def b_body(eatl_ref, eatr_ref, rowl_ref, coll_ref, rowr_ref, colr_ref,
               wbig_ref, bsel_ref, gidxl_ref, gidxr_ref):
        i = pl.program_id(0)
        dn = (((0,), (1,)), ((), ()))
        wb = wbig_ref[...]
        aml, gl = _bsel_half(eatl_ref[...], rowl_ref[i % 8, :],
                             coll_ref[i % 8, :])
        amr, gr = _bsel_half(eatr_ref[...], rowr_ref[i % 8, :],
                             colr_ref[i % 8, :])
        bl = lax.dot_general(aml, wb, dn, preferred_element_type=jnp.float32)
        br = lax.dot_general(amr, wb, dn, preferred_element_type=jnp.float32)
        bsel_ref[...] = jnp.concatenate([bl, br], axis=1)
        gidxl_ref[i % 8, :] = gl
        gidxr_ref[i % 8, :] = gr

    phase_b = pl.pallas_call(
        b_body,
        grid=(nb,),
        in_specs=[pl.BlockSpec((DE, eblk), lambda i: (0, i)),
                  pl.BlockSpec((DE, eblk), lambda i: (0, i + nb)),
                  pl.BlockSpec((8, eblk), lambda i: (i // 8, 0)),
                  pl.BlockSpec((8, eblk), lambda i: (i // 8, 0)),
                  pl.BlockSpec((8, eblk), lambda i: (i // 8, 0)),
                  pl.BlockSpec((8, eblk), lambda i: (i // 8, 0)),
                  pl.BlockSpec((H, 2 * DE), lambda i: (0, 0))],
        out_specs=[pl.BlockSpec((eblk, 2 * H), lambda i: (i, 0)),
                   pl.BlockSpec((8, eblk), lambda i: (i // 8, 0)),
                   pl.BlockSpec((8, eblk), lambda i: (i // 8, 0))],
        out_shape=[jax.ShapeDtypeStruct((eh, 2 * H), jnp.float32),
                   jax.ShapeDtypeStruct((nbp, eblk), jnp.int32),
                   jax.ShapeDtypeStruct((nbp, eblk), jnp.int32)],
    )

    def c_body(p1_ref, p2_ref, w1_ref, w2_ref, bn_ref, o_ref):
        fi = p1_ref[0] + p1_ref[1]
        fo = p2_ref[0] + p2_ref[1]
        dn = (((1,), (1,)), ((), ()))
        o = lax.dot_general(fi, w1_ref[...], dn,
                            preferred_element_type=jnp.float32)
        o = o + lax.dot_general(fo, w2_ref[...], dn,
                                preferred_element_type=jnp.float32)
        o_ref[...] = jnp.maximum(o + bn_ref[...], 0.0)

    nsteps = n // nblk
    phase_c = pl.pallas_call(
        c_body,
        grid=(nsteps,),
        in_specs=[pl.BlockSpec((2, nblk, H), lambda i: (0, i, 0)),
                  pl.BlockSpec((2, nblk, H), lambda i: (0, i + nsteps, 0)),
                  pl.BlockSpec((D, H), lambda i: (0, 0)),
                  pl.BlockSpec((D, H), lambda i: (0, 0)),
                  pl.BlockSpec((1, D), lambda i: (0, 0))],
        out_specs=pl.BlockSpec((nblk, D), lambda i: (i, 0)),
        out_shape=jax.ShapeDtypeStruct((n, D), jnp.float32),
    )

    return phase_a, phase_b, phase_c


@functools.lru_cache(maxsize=None)
def _build_sc(n, e):
    ew = e // WORKERS
    bk = 80
    while ew % bk:
        bk -= 8
    nbk = ew // bk
    cs = 160
    nchunk = (2 * n) // cs

    mesh = plsc.VectorSubcoreMesh(core_axis_name="c", subcore_axis_name="s",
                                  num_cores=2, num_subcores=16)

    @functools.partial(
        pl.kernel, mesh=mesh,
        compiler_params=pltpu.CompilerParams(use_tc_tiling_on_sc=False),
        out_type=jax.ShapeDtypeStruct((2, 2 * n, H), jnp.float32),
        scratch_types=[
            pltpu.VMEM((nbk, bk), jnp.int32),
            pltpu.VMEM((bk, H), jnp.float32),
            pltpu.VMEM((bk, H), jnp.float32),
            pltpu.VMEM((bk // 2, 2 * H), jnp.float32),
            pltpu.VMEM((bk // 2, 2 * H), jnp.float32),
            pltpu.VMEM((bk, H), jnp.float32),
            pltpu.VMEM((bk, H), jnp.float32),
            pltpu.VMEM((cs, H), jnp.float32),
            pltpu.VMEM_SHARED((2 * n + 8, H), jnp.float32),
            pltpu.SemaphoreType.DMA,
            pltpu.SemaphoreType.DMA,
            pltpu.SemaphoreType.DMA,
            pltpu.SemaphoreType.DMA,
            pltpu.SemaphoreType.DMA,
            pltpu.SemaphoreType.DMA,
        ],
    )
    def phase_sc(t_hbm, b_hbm, idx_hbm, z_hbm, out_hbm,
                 idx_all, g0, g1, b0, b1, r0, r1, bounce_v, acc_sh,
                 sg0, sg1, sb0, sb1, ss0, ss1):
        gath = (g0, g1)
        bsel = (b0, b1)
        res = (r0, r1)
        semg = (sg0, sg1)
        semb = (sb0, sb1)
        sems = (ss0, ss1)
        cid = lax.axis_index("c")
        sid = lax.axis_index("s")
        wid = sid * 2 + cid

        pltpu.sync_copy(idx_hbm.at[pl.ds(wid * nbk, nbk)], idx_all)

        pltpu.sync_copy(z_hbm, bounce_v)

        def zchunk(k, carry):
            @pl.when(sid == k % 16)
            def _():
                pltpu.sync_copy(bounce_v, acc_sh.at[pl.ds(k * cs, cs)])
            return carry

        lax.fori_loop(0, nchunk, zchunk, 0)
        plsc.subcore_barrier()

        def issue(s, p):
            base = (wid * ew + s * bk) // 2
            pltpu.async_copy(b_hbm.at[pl.ds(base, bk // 2)], bsel[p],
                             semb[p])
            pltpu.async_copy(t_hbm.at[idx_all.at[s]], gath[p], semg[p])

        def process(s, p):
            pltpu.make_async_copy(b_hbm.at[pl.ds(0, bk // 2)], bsel[p],
                                  semb[p]).wait()
            pltpu.make_async_copy(t_hbm.at[pl.ds(0, bk)], gath[p],
                                  semg[p]).wait()

            @pl.when(s >= 2)
            def _():
                pltpu.make_async_copy(t_hbm.at[pl.ds(0, bk)], res[p],
                                      sems[p]).wait()

            @plsc.parallel_loop(0, bk // 2, unroll=4)
            def _(ei):
                for j in range(H // LANES):
                    sl = pl.ds(j * LANES, LANES)
                    slh = pl.ds(H + j * LANES, LANES)
                    res[p][ei, sl] = jnp.maximum(
                        gath[p][ei, sl] + bsel[p][ei, sl], 0.0)
                    res[p][bk // 2 + ei, sl] = jnp.maximum(
                        gath[p][bk // 2 + ei, sl] + bsel[p][ei, slh], 0.0)
            pltpu.async_copy(res[p], acc_sh.at[idx_all.at[s]], sems[p],
                             add=True)

            @pl.when(s + 2 < nbk)
            def _():
                issue(s + 2, p)

        issue(0, 0)
        if nbk > 1:
            issue(1, 1)

        def pair(j, carry):
            for p in range(2):
                s = 2 * j + p

                @pl.when(s < nbk)
                def _():
                    process(s, p)
            return carry

        lax.fori_loop(0, (nbk + 1) // 2, pair, 0)
        pltpu.make_async_copy(t_hbm.at[pl.ds(0, bk)], res[0], sems[0]).wait()
        if nbk > 1:
            pltpu.make_async_copy(t_hbm.at[pl.ds(0, bk)], res[1],
                                  sems[1]).wait()
        plsc.subcore_barrier()

        def wchunk(k, carry):
            @pl.when(sid == k % 16)
            def _():
                pltpu.sync_copy(acc_sh.at[pl.ds(k * cs, cs)], bounce_v)
                pltpu.sync_copy(bounce_v, out_hbm.at[cid, pl.ds(k * cs, cs)])
            return carry

        lax.fori_loop(0, nchunk, wchunk, 0)

    return phase_sc, cs, bk


def kernel(x, edge_index, edge_attr, W_out, b_out, W_in, b_in, W_node,
           b_node):
    n = x.shape[0]
    e = edge_attr.shape[0]
    phase_a, phase_b, phase_c = _build(n, e)
    phase_sc, cs, bk = _build_sc(n, e)

    row = edge_index[0]
    col = edge_index[1]

    t = phase_a(x, W_in[:, :D], W_out[:, :D], b_in.reshape(1, H),
                b_out.reshape(1, H)).reshape(2 * n, H)
    t = jnp.concatenate([t, jnp.full((8, H), NEG, jnp.float32)], axis=0)
    wbig = jnp.concatenate([W_out[:, D:], W_in[:, D:]], axis=1)
    eh = e // 2
    eblk = 3200 if eh % 3200 == 0 else eh
    nb = eh // eblk
    nbp = ((nb + 7) // 8) * 8

    def chunk2(v):
        return jnp.pad(v.reshape(nb, eblk), ((0, nbp - nb), (0, 0)))

    eat = edge_attr.T
    bsel2, gl2, gr2 = phase_b(eat, eat, chunk2(row[:eh]), chunk2(col[:eh]),
                              chunk2(row[eh:]), chunk2(col[eh:]), wbig)
    hbk = bk // 2
    gidx_sc = jnp.concatenate([gl2[:nb].reshape(eh // hbk, hbk),
                               gr2[:nb].reshape(eh // hbk, hbk)], axis=1)
    zeros = jnp.zeros((cs, H), jnp.float32)
    partials = phase_sc(t, bsel2, gidx_sc, zeros)
    return phase_c(partials, partials, W_node[:, :H], W_node[:, H:],
                   b_node.reshape(1, D))
